# baseline (speedup 1.0000x reference)
import functools

import jax
import jax.numpy as jnp
from jax import lax
from jax.experimental import pallas as pl
from jax.experimental.pallas import tpu as pltpu
from jax.experimental.pallas import tpu_sc as plsc

VOCAB = 1000000
DIM = 32
B = 1024
K = 20
RW = 128

NC = 2
NS = 16
NW = NC * NS
BPW = B // NW
NPW = B * K // NW
NCHUNK = NPW // 128

TW = 8192


def _tc_transpose_body(in_ref, out_ref):
    i = lax.broadcasted_iota(jnp.int32, (DIM, DIM), 0)
    j = lax.broadcasted_iota(jnp.int32, (DIM, DIM), 1)
    eye = (i == j).astype(jnp.float32)
    out_ref[:, pl.ds(0, DIM)] = lax.dot_general(
        in_ref[...], eye, (((0,), (0,)), ((), ())),
        preferred_element_type=jnp.float32)


def _tc_transpose(embT):
    grid = (VOCAB + TW - 1) // TW
    return pl.pallas_call(
        _tc_transpose_body,
        grid=(grid,),
        in_specs=[pl.BlockSpec((DIM, TW), lambda c: (0, c))],
        out_specs=pl.BlockSpec((TW, RW), lambda c: (c, 0)),
        out_shape=jax.ShapeDtypeStruct((VOCAB, RW), jnp.float32),
    )(embT)


def _sc_gather_body(cidx_hbm, tidx_hbm, nidx_hbm, emb_hbm,
                    outc_hbm, outt_hbm, outn_hbm,
                    idx_c, idx_t, idx_n, rows_c, rows_t, rows_n, sem):
    w = lax.axis_index("s") * NC + lax.axis_index("c")
    pltpu.sync_copy(cidx_hbm.at[pl.ds(w * BPW, BPW)], idx_c)
    pltpu.sync_copy(tidx_hbm.at[pl.ds(w * BPW, BPW)], idx_t)
    for j in range(NCHUNK):
        pltpu.sync_copy(
            nidx_hbm.at[pl.ds(w * NPW + j * 128, 128)], idx_n[j])
    cps = [
        pltpu.async_copy(emb_hbm.at[idx_c], rows_c, sem),
        pltpu.async_copy(emb_hbm.at[idx_t], rows_t, sem),
    ]
    for j in range(NCHUNK):
        cps.append(pltpu.async_copy(
            emb_hbm.at[idx_n[j]],
            rows_n.at[pl.ds(j * 128, 128)], sem))
    for cp in cps:
        cp.wait()
    pltpu.sync_copy(rows_c, outc_hbm.at[pl.ds(w * BPW, BPW)])
    pltpu.sync_copy(rows_t, outt_hbm.at[pl.ds(w * BPW, BPW)])
    pltpu.sync_copy(rows_n, outn_hbm.at[pl.ds(w * NPW, NPW)])


_sc_gather = functools.partial(
    pl.kernel,
    out_type=(
        jax.ShapeDtypeStruct((B, RW), jnp.float32),
        jax.ShapeDtypeStruct((B, RW), jnp.float32),
        jax.ShapeDtypeStruct((B * K, RW), jnp.float32),
    ),
    mesh=plsc.VectorSubcoreMesh(core_axis_name="c", subcore_axis_name="s"),
    compiler_params=pltpu.CompilerParams(use_tc_tiling_on_sc=True),
    scratch_types=[
        pltpu.VMEM((BPW,), jnp.int32),
        pltpu.VMEM((BPW,), jnp.int32),
        [pltpu.VMEM((128,), jnp.int32) for _ in range(NCHUNK)],
        pltpu.VMEM((BPW, RW), jnp.float32),
        pltpu.VMEM((BPW, RW), jnp.float32),
        pltpu.VMEM((NPW, RW), jnp.float32),
        pltpu.SemaphoreType.DMA,
    ],
)(_sc_gather_body)


def _tc_loss_body(c_ref, t_ref, n_ref, out_ref):
    c = c_ref[:, pl.ds(0, DIM)]
    t = t_ref[:, pl.ds(0, DIM)]
    nsum = jnp.zeros((B, DIM), jnp.float32)
    for k in range(K):
        nsum = nsum + n_ref[pl.ds(k * B, B), pl.ds(0, DIM)]
    p = jnp.sum(t * c, axis=1, keepdims=True)
    n = -jnp.sum(nsum * c, axis=1, keepdims=True)
    loss = jax.nn.log_sigmoid(p) + jax.nn.log_sigmoid(n)
    out_ref[0, 0] = -jnp.mean(loss)


def kernel(center, target, noise, embeddings):
    center = center.astype(jnp.int32)
    target = target.astype(jnp.int32)
    nidx = jnp.transpose(noise.astype(jnp.int32)).reshape(B * K)
    emb_p = _tc_transpose(jnp.transpose(embeddings))
    c_rows, t_rows, n_rows = _sc_gather(center, target, nidx, emb_p)
    out = pl.pallas_call(
        _tc_loss_body,
        out_shape=jax.ShapeDtypeStruct((1, 1), jnp.float32),
        out_specs=pl.BlockSpec(memory_space=pltpu.SMEM),
    )(c_rows, t_rows, n_rows)
    return out[0, 0]


# speedup vs baseline: 1.0219x; 1.0219x over previous
import functools

import jax
import jax.numpy as jnp
from jax import lax
from jax.experimental import pallas as pl
from jax.experimental.pallas import tpu as pltpu
from jax.experimental.pallas import tpu_sc as plsc

VOCAB = 1000000
DIM = 32
B = 1024
K = 20
RW = 128

NC = 2
NS = 16
NW = NC * NS
BPW = B // NW
NPW = B * K // NW
NCHUNK = NPW // 128

TW = 8192


def _tc_transpose_body(in_ref, out_ref):
    out_ref[:, pl.ds(0, DIM)] = in_ref[...].T


def _tc_transpose(embT):
    grid = (VOCAB + TW - 1) // TW
    return pl.pallas_call(
        _tc_transpose_body,
        grid=(grid,),
        in_specs=[pl.BlockSpec((DIM, TW), lambda c: (0, c))],
        out_specs=pl.BlockSpec((TW, RW), lambda c: (c, 0)),
        out_shape=jax.ShapeDtypeStruct((VOCAB, RW), jnp.float32),
    )(embT)


def _sc_gather_body(cidx_hbm, tidx_hbm, nidx_hbm, emb_hbm,
                    outc_hbm, outt_hbm, outn_hbm,
                    idx_c, idx_t, idx_n, rows_c, rows_t, rows_n, sem):
    w = lax.axis_index("s") * NC + lax.axis_index("c")
    pltpu.sync_copy(cidx_hbm.at[pl.ds(w * BPW, BPW)], idx_c)
    pltpu.sync_copy(tidx_hbm.at[pl.ds(w * BPW, BPW)], idx_t)
    for j in range(NCHUNK):
        pltpu.sync_copy(
            nidx_hbm.at[pl.ds(w * NPW + j * 128, 128)], idx_n[j])
    cps = [
        pltpu.async_copy(emb_hbm.at[idx_c], rows_c, sem),
        pltpu.async_copy(emb_hbm.at[idx_t], rows_t, sem),
    ]
    for j in range(NCHUNK):
        cps.append(pltpu.async_copy(
            emb_hbm.at[idx_n[j]],
            rows_n.at[pl.ds(j * 128, 128)], sem))
    for cp in cps:
        cp.wait()
    pltpu.sync_copy(rows_c, outc_hbm.at[pl.ds(w * BPW, BPW)])
    pltpu.sync_copy(rows_t, outt_hbm.at[pl.ds(w * BPW, BPW)])
    pltpu.sync_copy(rows_n, outn_hbm.at[pl.ds(w * NPW, NPW)])


_sc_gather = functools.partial(
    pl.kernel,
    out_type=(
        jax.ShapeDtypeStruct((B, RW), jnp.float32),
        jax.ShapeDtypeStruct((B, RW), jnp.float32),
        jax.ShapeDtypeStruct((B * K, RW), jnp.float32),
    ),
    mesh=plsc.VectorSubcoreMesh(core_axis_name="c", subcore_axis_name="s"),
    compiler_params=pltpu.CompilerParams(use_tc_tiling_on_sc=True),
    scratch_types=[
        pltpu.VMEM((BPW,), jnp.int32),
        pltpu.VMEM((BPW,), jnp.int32),
        [pltpu.VMEM((128,), jnp.int32) for _ in range(NCHUNK)],
        pltpu.VMEM((BPW, RW), jnp.float32),
        pltpu.VMEM((BPW, RW), jnp.float32),
        pltpu.VMEM((NPW, RW), jnp.float32),
        pltpu.SemaphoreType.DMA,
    ],
)(_sc_gather_body)


def _tc_loss_body(c_ref, t_ref, n_ref, out_ref):
    c = c_ref[:, pl.ds(0, DIM)]
    t = t_ref[:, pl.ds(0, DIM)]
    nsum = jnp.zeros((B, DIM), jnp.float32)
    for k in range(K):
        nsum = nsum + n_ref[pl.ds(k * B, B), pl.ds(0, DIM)]
    p = jnp.sum(t * c, axis=1, keepdims=True)
    n = -jnp.sum(nsum * c, axis=1, keepdims=True)
    loss = jax.nn.log_sigmoid(p) + jax.nn.log_sigmoid(n)
    out_ref[0, 0] = -jnp.mean(loss)


def kernel(center, target, noise, embeddings):
    center = center.astype(jnp.int32)
    target = target.astype(jnp.int32)
    nidx = jnp.transpose(noise.astype(jnp.int32)).reshape(B * K)
    emb_p = _tc_transpose(jnp.transpose(embeddings))
    c_rows, t_rows, n_rows = _sc_gather(center, target, nidx, emb_p)
    out = pl.pallas_call(
        _tc_loss_body,
        out_shape=jax.ShapeDtypeStruct((1, 1), jnp.float32),
        out_specs=pl.BlockSpec(memory_space=pltpu.SMEM),
    )(c_rows, t_rows, n_rows)
    return out[0, 0]
